# baseline (speedup 1.0000x reference)
import functools

import jax
import jax.numpy as jnp
from jax import lax
from jax.experimental import pallas as pl
from jax.experimental.pallas import tpu as pltpu
from jax.experimental.pallas import tpu_sc as plsc

NN = 10000
EE = 320000
DD = 64
NC = 2
NS = 16
NW = NC * NS
EPT = EE // NW
CH = 400
NCHUNK = EPT // CH
SUB = 80
NSUB = CH // SUB
WROW = 80
CHC = 80
NCHC = EPT // CHC

_MESH = plsc.VectorSubcoreMesh(core_axis_name="c", subcore_axis_name="s")
_SC_PARAMS = pltpu.CompilerParams(use_tc_tiling_on_sc=False,
                                  needs_layout_passes=False)


def _wid():
    return lax.axis_index("s") * NC + lax.axis_index("c")


@functools.partial(
    pl.kernel,
    out_type=jax.ShapeDtypeStruct((EE,), jnp.float32),
    mesh=_MESH,
    compiler_params=_SC_PARAMS,
    scratch_types=[
        pltpu.VMEM((NN,), jnp.float32),
        pltpu.VMEM((NN,), jnp.float32),
        pltpu.VMEM((NN,), jnp.float32),
        pltpu.VMEM((CH,), jnp.int32),
        pltpu.VMEM((CH,), jnp.int32),
        pltpu.VMEM((CH,), jnp.float32),
    ],
)
def _sc_d2(px_hbm, py_hbm, pz_hbm, src_hbm, dst_hbm, d2_hbm,
           px, py, pz, sidx, didx, d2v):
    base = _wid() * EPT
    pltpu.sync_copy(px_hbm, px)
    pltpu.sync_copy(py_hbm, py)
    pltpu.sync_copy(pz_hbm, pz)

    def chunk_body(ci, carry):
        off = base + ci * CH
        pltpu.sync_copy(src_hbm.at[pl.ds(off, CH)], sidx)
        pltpu.sync_copy(dst_hbm.at[pl.ds(off, CH)], didx)

        def grp(gi, c2):
            s = sidx[pl.ds(gi * 16, 16)]
            d = didx[pl.ds(gi * 16, 16)]
            acc = jnp.zeros((16,), jnp.float32)
            for ref in (px, py, pz):
                dif = plsc.load_gather(ref, [d]) - plsc.load_gather(ref, [s])
                acc = acc + dif * dif
            d2v[pl.ds(gi * 16, 16)] = acc
            return c2

        lax.fori_loop(0, CH // 16, grp, 0)
        pltpu.sync_copy(d2v, d2_hbm.at[pl.ds(off, CH)])
        return carry

    lax.fori_loop(0, NCHUNK, chunk_body, 0)


@functools.partial(
    pl.kernel,
    out_type=(jax.ShapeDtypeStruct((EE,), jnp.float32),
              jax.ShapeDtypeStruct((NW, 16), jnp.float32)),
    mesh=_MESH,
    compiler_params=_SC_PARAMS,
    scratch_types=[
        pltpu.VMEM((NSUB, SUB), jnp.int32),
        pltpu.VMEM((NSUB, SUB), jnp.int32),
        pltpu.VMEM((CH, DD), jnp.float32),
        pltpu.VMEM((CH, DD), jnp.float32),
        pltpu.VMEM((CH, DD), jnp.float32),
        pltpu.VMEM((CH,), jnp.float32),
        pltpu.VMEM((16,), jnp.float32),
        pltpu.SemaphoreType.DMA,
    ],
)
def _sc_passA(q_hbm, k_hbm, rad_hbm, src2_hbm, dst2_hbm,
              logits_hbm, gmax_hbm,
              sidx2, didx2, qrows, krows, radc, lbuf, gmaxv, sem):
    wid = _wid()
    base = wid * EPT
    lane = lax.broadcasted_iota(jnp.int32, (16,), 0)

    def chunk_body(ci, lmax):
        off = base + ci * CH
        row0 = off // SUB
        pltpu.sync_copy(src2_hbm.at[pl.ds(row0, NSUB)], sidx2)
        pltpu.sync_copy(dst2_hbm.at[pl.ds(row0, NSUB)], didx2)
        descs = []
        for j in range(NSUB):
            descs.append(pltpu.async_copy(
                q_hbm.at[didx2.at[j]], qrows.at[pl.ds(j * SUB, SUB)], sem))
            descs.append(pltpu.async_copy(
                k_hbm.at[sidx2.at[j]], krows.at[pl.ds(j * SUB, SUB)], sem))
        pltpu.sync_copy(rad_hbm.at[pl.ds(off, CH)], radc)
        for dsc in descs:
            dsc.wait()

        def grp(gi, lm):
            ev = lane + gi * 16
            acc = jnp.zeros((16,), jnp.float32)
            for d in range(DD):
                dv = jnp.full((16,), d, jnp.int32)
                acc = acc + (plsc.load_gather(qrows, [ev, dv])
                             * plsc.load_gather(krows, [ev, dv])
                             * plsc.load_gather(radc, [ev, dv]))
            lgvec = acc * 0.125
            lbuf[pl.ds(gi * 16, 16)] = lgvec
            return jnp.maximum(lm, lgvec)

        lmax = lax.fori_loop(0, CH // 16, grp, lmax)
        pltpu.sync_copy(lbuf, logits_hbm.at[pl.ds(off, CH)])
        return lmax

    lmax0 = jnp.full((16,), -3.0e38, jnp.float32)
    lmax = lax.fori_loop(0, NCHUNK, chunk_body, lmax0)
    gmaxv[...] = jnp.full((16,), jnp.max(lmax), jnp.float32)
    pltpu.sync_copy(gmaxv, gmax_hbm.at[wid])


@functools.partial(
    pl.kernel,
    out_type=jax.ShapeDtypeStruct((NC, NN, WROW), jnp.float32),
    mesh=_MESH,
    compiler_params=_SC_PARAMS,
    scratch_types=[
        pltpu.VMEM((1, CHC), jnp.int32),
        pltpu.VMEM((1, CHC), jnp.int32),
        pltpu.VMEM((CHC, DD), jnp.float32),
        pltpu.VMEM((CHC, DD), jnp.float32),
        pltpu.VMEM((CHC,), jnp.float32),
        pltpu.VMEM((CHC,), jnp.float32),
        pltpu.VMEM((CHC, WROW), jnp.float32),
        pltpu.VMEM((125, WROW), jnp.float32),
        pltpu.VMEM((NW, 16), jnp.float32),
        pltpu.VMEM_SHARED((NN, WROW), jnp.float32),
        pltpu.SemaphoreType.DMA,
    ],
)
def _sc_passC(v_hbm, rad_hbm, logits_hbm, gmax_hbm, src2_hbm, dst2_hbm,
              agg_hbm,
              sidx2, didx2, vrows, radc, lbuf, exbuf, contrib, zbuf, gmv,
              agg_sh, sem):
    cid = lax.axis_index("c")
    sid = lax.axis_index("s")
    wid = _wid()
    base = wid * EPT
    lane = lax.broadcasted_iota(jnp.int32, (16,), 0)
    zv = jnp.zeros((16,), jnp.float32)

    pltpu.sync_copy(gmax_hbm, gmv)
    mv = gmv[0]
    for kk in range(1, NW):
        mv = jnp.maximum(mv, gmv[kk])
    c = jnp.max(mv)

    def zrow(ri, carry):
        for j in range(WROW // 16):
            zbuf[ri, pl.ds(j * 16, 16)] = zv
        return carry
    lax.fori_loop(0, 125, zrow, 0)
    for t in range(5):
        pltpu.sync_copy(zbuf, agg_sh.at[pl.ds(sid * 625 + t * 125, 125)])
    plsc.subcore_barrier()

    def chunk_body(ci, carry):
        off = base + ci * CHC
        row0 = off // SUB
        pltpu.sync_copy(src2_hbm.at[pl.ds(row0, 1)], sidx2)
        pltpu.sync_copy(dst2_hbm.at[pl.ds(row0, 1)], didx2)
        dsc = pltpu.async_copy(v_hbm.at[sidx2.at[0]], vrows, sem)
        pltpu.sync_copy(rad_hbm.at[pl.ds(off, CHC)], radc)
        pltpu.sync_copy(logits_hbm.at[pl.ds(off, CHC)], lbuf)
        dsc.wait()

        def grp(gi, c2):
            exbuf[pl.ds(gi * 16, 16)] = jnp.exp(
                lbuf[pl.ds(gi * 16, 16)] - c)
            return c2
        lax.fori_loop(0, CHC // 16, grp, 0)

        def edge(r, c2):
            exv = plsc.load_gather(exbuf, [jnp.broadcast_to(r, (16,))])
            for j in range(DD // 16):
                contrib[r, pl.ds(j * 16, 16)] = (
                    exv * vrows[r, pl.ds(j * 16, 16)]
                    * radc[r, pl.ds(j * 16, 16)])
            contrib[r, pl.ds(DD, 16)] = jnp.where(lane == 0, exv, 0.0)
            return c2
        lax.fori_loop(0, CHC, edge, 0)

        pltpu.sync_copy(contrib, agg_sh.at[didx2.at[0]], add=True)
        return carry

    lax.fori_loop(0, NCHC, chunk_body, 0)
    plsc.subcore_barrier()

    @pl.when(sid == 0)
    def _():
        pltpu.sync_copy(agg_sh, agg_hbm.at[cid])


@functools.partial(
    pl.kernel,
    out_type=jax.ShapeDtypeStruct((NC, NN, DD), jnp.float32),
    mesh=_MESH,
    compiler_params=_SC_PARAMS,
    scratch_types=[
        pltpu.VMEM((1, CHC), jnp.int32),
        pltpu.VMEM((1, CHC), jnp.int32),
        pltpu.VMEM((CHC, DD), jnp.float32),
        pltpu.VMEM((CHC, DD), jnp.float32),
        pltpu.VMEM((CHC, DD), jnp.float32),
        pltpu.VMEM((125, DD), jnp.float32),
        pltpu.VMEM_SHARED((NN, DD), jnp.float32),
        pltpu.SemaphoreType.DMA,
    ],
)
def _sc_passD(hc_hbm, rad_hbm, src2_hbm, dst2_hbm, agg_hbm,
              sidx2, didx2, hcrows, radc, contrib, zbuf, agg_sh, sem):
    cid = lax.axis_index("c")
    sid = lax.axis_index("s")
    base = _wid() * EPT
    zv = jnp.zeros((16,), jnp.float32)

    def zrow(ri, carry):
        for j in range(DD // 16):
            zbuf[ri, pl.ds(j * 16, 16)] = zv
        return carry
    lax.fori_loop(0, 125, zrow, 0)
    for t in range(5):
        pltpu.sync_copy(zbuf, agg_sh.at[pl.ds(sid * 625 + t * 125, 125)])
    plsc.subcore_barrier()

    def chunk_body(ci, carry):
        off = base + ci * CHC
        row0 = off // SUB
        pltpu.sync_copy(src2_hbm.at[pl.ds(row0, 1)], sidx2)
        pltpu.sync_copy(dst2_hbm.at[pl.ds(row0, 1)], didx2)
        dsc = pltpu.async_copy(hc_hbm.at[sidx2.at[0]], hcrows, sem)
        pltpu.sync_copy(rad_hbm.at[pl.ds(off, CHC)], radc)
        dsc.wait()

        def edge(r, c2):
            for j in range(DD // 16):
                contrib[r, pl.ds(j * 16, 16)] = (
                    hcrows[r, pl.ds(j * 16, 16)]
                    * radc[r, pl.ds(j * 16, 16)])
            return c2
        lax.fori_loop(0, CHC, edge, 0)

        pltpu.sync_copy(contrib, agg_sh.at[didx2.at[0]], add=True)
        return carry

    lax.fori_loop(0, NCHC, chunk_body, 0)
    plsc.subcore_barrier()

    @pl.when(sid == 0)
    def _():
        pltpu.sync_copy(agg_sh, agg_hbm.at[cid])


def _tc_prologue_body(at_ref, emb_ref, win_ref, h_ref):
    at = at_ref[0, 0, :]
    onehot = (at[:, None]
              == lax.broadcasted_iota(jnp.int32, (1000, 100), 1)
              ).astype(jnp.float32)
    table = jnp.dot(emb_ref[...], win_ref[...],
                    preferred_element_type=jnp.float32)
    h_ref[...] = jnp.dot(onehot, table, preferred_element_type=jnp.float32)


def _tc_prologue(atomic3, emb, win):
    return pl.pallas_call(
        _tc_prologue_body,
        grid=(10,),
        in_specs=[
            pl.BlockSpec((1, 1, 1000), lambda i: (i, 0, 0)),
            pl.BlockSpec((100, 128), lambda i: (0, 0)),
            pl.BlockSpec((128, DD), lambda i: (0, 0)),
        ],
        out_specs=pl.BlockSpec((1000, DD), lambda i: (i, 0)),
        out_shape=jax.ShapeDtypeStruct((NN, DD), jnp.float32),
    )(atomic3, emb, win)


def _tc_rad_body(d2_ref, w1_ref, w2_ref, rad_ref):
    r = jnp.sqrt(d2_ref[0, 0, :] + 1e-8)
    a = jnp.maximum(r[:, None] * w1_ref[0, 0, :][None, :], 0.0)
    rad_ref[0] = jnp.dot(a, w2_ref[0], preferred_element_type=jnp.float32)


def _tc_rad(d2_3, wr1s, wr2s):
    return pl.pallas_call(
        _tc_rad_body,
        grid=(5, 80),
        in_specs=[
            pl.BlockSpec((1, 1, 4000), lambda i, j: (j, 0, 0)),
            pl.BlockSpec((1, 1, 16), lambda i, j: (i, 0, 0)),
            pl.BlockSpec((1, 16, DD), lambda i, j: (i, 0, 0)),
        ],
        out_specs=pl.BlockSpec((1, 4000, DD), lambda i, j: (i, j, 0)),
        out_shape=jax.ShapeDtypeStruct((5, EE, DD), jnp.float32),
    )(d2_3, wr1s, wr2s)


def _tc_proj_body(h_ref, wq_ref, wk_ref, wv_ref, q_ref, k_ref, v_ref):
    h = h_ref[...]
    q_ref[...] = jnp.dot(h, wq_ref[...], preferred_element_type=jnp.float32)
    k_ref[...] = jnp.dot(h, wk_ref[...], preferred_element_type=jnp.float32)
    v_ref[...] = jnp.dot(h, wv_ref[...], preferred_element_type=jnp.float32)


def _tc_proj(h, wq, wk, wv):
    blk = pl.BlockSpec((1000, DD), lambda i: (i, 0))
    wblk = pl.BlockSpec((DD, DD), lambda i: (0, 0))
    sds = jax.ShapeDtypeStruct((NN, DD), jnp.float32)
    return pl.pallas_call(
        _tc_proj_body,
        grid=(10,),
        in_specs=[blk, wblk, wblk, wblk],
        out_specs=(blk, blk, blk),
        out_shape=(sds, sds, sds),
    )(h, wq, wk, wv)


def _norm_update(h, aggb, grow, brow):
    numer = aggb[0, :, 0:DD] + aggb[1, :, 0:DD]
    den = aggb[0, :, DD:DD + 1] + aggb[1, :, DD:DD + 1]
    hn = h + numer / (den + 1e-9)
    nrm = jnp.abs(hn) + 1e-9
    return (hn / nrm) * jnp.maximum(nrm * grow + brow, 0.0)


def _tc_update_proj_body(h_ref, agg_ref, g_ref, b_ref,
                         wq_ref, wk_ref, wv_ref,
                         h_out, q_ref, k_ref, v_ref):
    hn = _norm_update(h_ref[...], agg_ref[...], g_ref[...], b_ref[...])
    h_out[...] = hn
    q_ref[...] = jnp.dot(hn, wq_ref[...], preferred_element_type=jnp.float32)
    k_ref[...] = jnp.dot(hn, wk_ref[...], preferred_element_type=jnp.float32)
    v_ref[...] = jnp.dot(hn, wv_ref[...], preferred_element_type=jnp.float32)


def _tc_update_proj(h, agg, grow, brow, wq, wk, wv):
    blk = pl.BlockSpec((1000, DD), lambda i: (i, 0))
    wblk = pl.BlockSpec((DD, DD), lambda i: (0, 0))
    gblk = pl.BlockSpec((1, DD), lambda i: (0, 0))
    ablk = pl.BlockSpec((NC, 1000, WROW), lambda i: (0, i, 0))
    sds = jax.ShapeDtypeStruct((NN, DD), jnp.float32)
    return pl.pallas_call(
        _tc_update_proj_body,
        grid=(10,),
        in_specs=[blk, ablk, gblk, gblk, wblk, wblk, wblk],
        out_specs=(blk, blk, blk, blk),
        out_shape=(sds, sds, sds, sds),
    )(h, agg, grow, brow, wq, wk, wv)


def _tc_update_proj2_body(h_ref, agg_ref, g_ref, b_ref, wc_ref, ws_ref,
                          hc_ref, hs_ref):
    hn = _norm_update(h_ref[...], agg_ref[...], g_ref[...], b_ref[...])
    hc_ref[...] = jnp.dot(hn, wc_ref[...], preferred_element_type=jnp.float32)
    hs_ref[...] = jnp.dot(hn, ws_ref[...], preferred_element_type=jnp.float32)


def _tc_update_proj2(h, agg, grow, brow, wc, wself):
    blk = pl.BlockSpec((1000, DD), lambda i: (i, 0))
    wblk = pl.BlockSpec((DD, DD), lambda i: (0, 0))
    gblk = pl.BlockSpec((1, DD), lambda i: (0, 0))
    ablk = pl.BlockSpec((NC, 1000, WROW), lambda i: (0, i, 0))
    sds = jax.ShapeDtypeStruct((NN, DD), jnp.float32)
    return pl.pallas_call(
        _tc_update_proj2_body,
        grid=(10,),
        in_specs=[blk, ablk, gblk, gblk, wblk, wblk],
        out_specs=(blk, blk),
        out_shape=(sds, sds),
    )(h, agg, grow, brow, wc, wself)


def _tc_final_body(conv_ref, hs_ref, w1_ref, b1_ref, w2_ref, b2_ref, out_ref):
    h2 = conv_ref[0] + conv_ref[1] + hs_ref[...]
    hp = jnp.mean(h2, axis=0, keepdims=True)
    z = jnp.maximum(
        jnp.dot(hp, w1_ref[...], preferred_element_type=jnp.float32)
        + b1_ref[...], 0.0)
    out_ref[...] = (jnp.dot(z, w2_ref[...],
                            preferred_element_type=jnp.float32)
                    + b2_ref[...])


def _tc_final(conv, hself, w1, b1, w2, b2):
    return pl.pallas_call(
        _tc_final_body,
        out_shape=jax.ShapeDtypeStruct((1, 1), jnp.float32),
    )(conv, hself, w1, b1, w2, b2)


def kernel(pos, atomic_numbers, edge_index, emb, Win, Wq, Wk, Wv, Wr1, Wr2,
           g, b, Wc, Wrc1, Wrc2, Wself, W1, b1, W2, b2):
    src = edge_index[0].astype(jnp.int32)
    dst = edge_index[1].astype(jnp.int32)
    src2 = src.reshape(EE // SUB, SUB)
    dst2 = dst.reshape(EE // SUB, SUB)
    atomic3 = atomic_numbers.astype(jnp.int32).reshape(10, 1, 1000)

    d2 = _sc_d2(pos[:, 0] + 0.0, pos[:, 1] + 0.0, pos[:, 2] + 0.0, src, dst)

    h = _tc_prologue(atomic3, emb, Win)

    wr1s = jnp.concatenate([Wr1[:, 0, :], Wrc1], axis=0).reshape(5, 1, 16)
    wr2s = jnp.concatenate([Wr2, Wrc2[None]], axis=0)
    rad_all = _tc_rad(d2.reshape(80, 1, 4000), wr1s, wr2s)

    agg = None
    for i in range(4):
        if i == 0:
            q, kf, vf = _tc_proj(h, Wq[0], Wk[0], Wv[0])
        else:
            h, q, kf, vf = _tc_update_proj(
                h, agg, g[i - 1].reshape(1, DD), b[i - 1].reshape(1, DD),
                Wq[i], Wk[i], Wv[i])
        logits, gmax = _sc_passA(q, kf, rad_all[i], src2, dst2)
        agg = _sc_passC(vf, rad_all[i], logits, gmax, src2, dst2)

    hc, hself = _tc_update_proj2(
        h, agg, g[3].reshape(1, DD), b[3].reshape(1, DD), Wc, Wself)
    conv = _sc_passD(hc, rad_all[4], src2, dst2)
    return _tc_final(conv, hself, W1, b1.reshape(1, DD), W2,
                     b2.reshape(1, 1))


# speedup vs baseline: 1.6912x; 1.6912x over previous
import functools

import jax
import jax.numpy as jnp
from jax import lax
from jax.experimental import pallas as pl
from jax.experimental.pallas import tpu as pltpu
from jax.experimental.pallas import tpu_sc as plsc

NN = 10000
EE = 320000
DD = 64
NC = 2
NS = 16
NW = NC * NS
EPT = EE // NW
CH = 400
NCHUNK = EPT // CH
SUB = 80
NSUB = CH // SUB
WROW = 80
CHC = 80
NCHC = EPT // CHC
NBUF = 5

_MESH = plsc.VectorSubcoreMesh(core_axis_name="c", subcore_axis_name="s")
_SC_PARAMS = pltpu.CompilerParams(use_tc_tiling_on_sc=False,
                                  needs_layout_passes=False)


def _wid():
    return lax.axis_index("s") * NC + lax.axis_index("c")


@functools.partial(
    pl.kernel,
    out_type=jax.ShapeDtypeStruct((EE,), jnp.float32),
    mesh=_MESH,
    compiler_params=_SC_PARAMS,
    scratch_types=[
        pltpu.VMEM((NN,), jnp.float32),
        pltpu.VMEM((NN,), jnp.float32),
        pltpu.VMEM((NN,), jnp.float32),
        pltpu.VMEM((CH,), jnp.int32),
        pltpu.VMEM((CH,), jnp.int32),
        pltpu.VMEM((CH,), jnp.float32),
    ],
)
def _sc_d2(px_hbm, py_hbm, pz_hbm, src_hbm, dst_hbm, d2_hbm,
           px, py, pz, sidx, didx, d2v):
    base = _wid() * EPT
    pltpu.sync_copy(px_hbm, px)
    pltpu.sync_copy(py_hbm, py)
    pltpu.sync_copy(pz_hbm, pz)

    def chunk_body(ci, carry):
        off = base + ci * CH
        pltpu.sync_copy(src_hbm.at[pl.ds(off, CH)], sidx)
        pltpu.sync_copy(dst_hbm.at[pl.ds(off, CH)], didx)

        def grp(gi, c2):
            s = sidx[pl.ds(gi * 16, 16)]
            d = didx[pl.ds(gi * 16, 16)]
            acc = jnp.zeros((16,), jnp.float32)
            for ref in (px, py, pz):
                dif = plsc.load_gather(ref, [d]) - plsc.load_gather(ref, [s])
                acc = acc + dif * dif
            d2v[pl.ds(gi * 16, 16)] = acc
            return c2

        lax.fori_loop(0, CH // 16, grp, 0)
        pltpu.sync_copy(d2v, d2_hbm.at[pl.ds(off, CH)])
        return carry

    lax.fori_loop(0, NCHUNK, chunk_body, 0)


@functools.partial(
    pl.kernel,
    out_type=(jax.ShapeDtypeStruct((EE,), jnp.float32),
              jax.ShapeDtypeStruct((NW, 16), jnp.float32)),
    mesh=_MESH,
    compiler_params=_SC_PARAMS,
    scratch_types=[
        pltpu.VMEM((NCHC, CHC), jnp.int32),
        pltpu.VMEM((NCHC, CHC), jnp.int32),
        pltpu.VMEM((NBUF, CHC, DD), jnp.float32),
        pltpu.VMEM((NBUF, CHC, DD), jnp.float32),
        pltpu.VMEM((NBUF, CHC, DD), jnp.float32),
        pltpu.VMEM((NBUF, CHC), jnp.float32),
        pltpu.VMEM((16,), jnp.float32),
        [pltpu.SemaphoreType.DMA] * NBUF,
    ],
)
def _sc_passA(q_hbm, k_hbm, rad_hbm, src2_hbm, dst2_hbm,
              logits_hbm, gmax_hbm,
              sidxall, didxall, qrows, krows, radc, lbuf, gmaxv, sems):
    wid = _wid()
    base = wid * EPT
    lane = lax.broadcasted_iota(jnp.int32, (16,), 0)
    pltpu.sync_copy(src2_hbm.at[pl.ds(wid * NCHC, NCHC)], sidxall)
    pltpu.sync_copy(dst2_hbm.at[pl.ds(wid * NCHC, NCHC)], didxall)

    def issue(ci, bb):
        off = base + ci * CHC
        pltpu.async_copy(q_hbm.at[didxall.at[ci]], qrows.at[bb], sems[bb])
        pltpu.async_copy(k_hbm.at[sidxall.at[ci]], krows.at[bb], sems[bb])
        pltpu.async_copy(rad_hbm.at[pl.ds(off, CHC)], radc.at[bb], sems[bb])

    for bb in range(NBUF):
        issue(bb, bb)

    def outer(gg, lmax):
        for bb in range(NBUF):
            ci = gg * NBUF + bb
            off = base + ci * CHC
            pltpu.make_async_copy(
                q_hbm.at[didxall.at[ci]], qrows.at[bb], sems[bb]).wait()
            pltpu.make_async_copy(
                k_hbm.at[sidxall.at[ci]], krows.at[bb], sems[bb]).wait()
            pltpu.make_async_copy(
                rad_hbm.at[pl.ds(off, CHC)], radc.at[bb], sems[bb]).wait()

            def grp(gi, lm):
                lgvec = jnp.zeros((16,), jnp.float32)
                for t in range(16):
                    r = gi * 16 + t
                    acc = jnp.zeros((16,), jnp.float32)
                    for j in range(DD // 16):
                        acc = acc + (qrows[bb, r, pl.ds(j * 16, 16)]
                                     * krows[bb, r, pl.ds(j * 16, 16)]
                                     * radc[bb, r, pl.ds(j * 16, 16)])
                    lgvec = jnp.where(lane == t, jnp.sum(acc) * 0.125,
                                      lgvec)
                lbuf[bb, pl.ds(gi * 16, 16)] = lgvec
                return jnp.maximum(lm, lgvec)

            lmax = lax.fori_loop(0, CHC // 16, grp, lmax)
            pltpu.sync_copy(lbuf.at[bb], logits_hbm.at[pl.ds(off, CHC)])

            @pl.when(ci + NBUF < NCHC)
            def _():
                issue(ci + NBUF, bb)
        return lmax

    lmax0 = jnp.full((16,), -3.0e38, jnp.float32)
    lmax = lax.fori_loop(0, NCHC // NBUF, outer, lmax0)
    gmaxv[...] = jnp.full((16,), jnp.max(lmax), jnp.float32)
    pltpu.sync_copy(gmaxv, gmax_hbm.at[wid])


@functools.partial(
    pl.kernel,
    out_type=jax.ShapeDtypeStruct((NC, NN, WROW), jnp.float32),
    mesh=_MESH,
    compiler_params=_SC_PARAMS,
    scratch_types=[
        pltpu.VMEM((1, CHC), jnp.int32),
        pltpu.VMEM((1, CHC), jnp.int32),
        pltpu.VMEM((CHC, DD), jnp.float32),
        pltpu.VMEM((CHC, DD), jnp.float32),
        pltpu.VMEM((CHC,), jnp.float32),
        pltpu.VMEM((CHC,), jnp.float32),
        pltpu.VMEM((CHC, WROW), jnp.float32),
        pltpu.VMEM((125, WROW), jnp.float32),
        pltpu.VMEM((NW, 16), jnp.float32),
        pltpu.VMEM_SHARED((NN, WROW), jnp.float32),
        pltpu.SemaphoreType.DMA,
    ],
)
def _sc_passC(v_hbm, rad_hbm, logits_hbm, gmax_hbm, src2_hbm, dst2_hbm,
              agg_hbm,
              sidx2, didx2, vrows, radc, lbuf, exbuf, contrib, zbuf, gmv,
              agg_sh, sem):
    cid = lax.axis_index("c")
    sid = lax.axis_index("s")
    wid = _wid()
    base = wid * EPT
    lane = lax.broadcasted_iota(jnp.int32, (16,), 0)
    zv = jnp.zeros((16,), jnp.float32)

    pltpu.sync_copy(gmax_hbm, gmv)
    mv = gmv[0]
    for kk in range(1, NW):
        mv = jnp.maximum(mv, gmv[kk])
    c = jnp.max(mv)

    def zrow(ri, carry):
        for j in range(WROW // 16):
            zbuf[ri, pl.ds(j * 16, 16)] = zv
        return carry
    lax.fori_loop(0, 125, zrow, 0)
    for t in range(5):
        pltpu.sync_copy(zbuf, agg_sh.at[pl.ds(sid * 625 + t * 125, 125)])
    plsc.subcore_barrier()

    def chunk_body(ci, carry):
        off = base + ci * CHC
        row0 = off // SUB
        pltpu.sync_copy(src2_hbm.at[pl.ds(row0, 1)], sidx2)
        pltpu.sync_copy(dst2_hbm.at[pl.ds(row0, 1)], didx2)
        dsc = pltpu.async_copy(v_hbm.at[sidx2.at[0]], vrows, sem)
        pltpu.sync_copy(rad_hbm.at[pl.ds(off, CHC)], radc)
        pltpu.sync_copy(logits_hbm.at[pl.ds(off, CHC)], lbuf)
        dsc.wait()

        def grp(gi, c2):
            exbuf[pl.ds(gi * 16, 16)] = jnp.exp(
                lbuf[pl.ds(gi * 16, 16)] - c)
            return c2
        lax.fori_loop(0, CHC // 16, grp, 0)

        def edge(r, c2):
            exv = plsc.load_gather(exbuf, [jnp.broadcast_to(r, (16,))])
            for j in range(DD // 16):
                contrib[r, pl.ds(j * 16, 16)] = (
                    exv * vrows[r, pl.ds(j * 16, 16)]
                    * radc[r, pl.ds(j * 16, 16)])
            contrib[r, pl.ds(DD, 16)] = jnp.where(lane == 0, exv, 0.0)
            return c2
        lax.fori_loop(0, CHC, edge, 0)

        pltpu.sync_copy(contrib, agg_sh.at[didx2.at[0]], add=True)
        return carry

    lax.fori_loop(0, NCHC, chunk_body, 0)
    plsc.subcore_barrier()

    @pl.when(sid == 0)
    def _():
        pltpu.sync_copy(agg_sh, agg_hbm.at[cid])


@functools.partial(
    pl.kernel,
    out_type=jax.ShapeDtypeStruct((NC, NN, DD), jnp.float32),
    mesh=_MESH,
    compiler_params=_SC_PARAMS,
    scratch_types=[
        pltpu.VMEM((1, CHC), jnp.int32),
        pltpu.VMEM((1, CHC), jnp.int32),
        pltpu.VMEM((CHC, DD), jnp.float32),
        pltpu.VMEM((CHC, DD), jnp.float32),
        pltpu.VMEM((CHC, DD), jnp.float32),
        pltpu.VMEM((125, DD), jnp.float32),
        pltpu.VMEM_SHARED((NN, DD), jnp.float32),
        pltpu.SemaphoreType.DMA,
    ],
)
def _sc_passD(hc_hbm, rad_hbm, src2_hbm, dst2_hbm, agg_hbm,
              sidx2, didx2, hcrows, radc, contrib, zbuf, agg_sh, sem):
    cid = lax.axis_index("c")
    sid = lax.axis_index("s")
    base = _wid() * EPT
    zv = jnp.zeros((16,), jnp.float32)

    def zrow(ri, carry):
        for j in range(DD // 16):
            zbuf[ri, pl.ds(j * 16, 16)] = zv
        return carry
    lax.fori_loop(0, 125, zrow, 0)
    for t in range(5):
        pltpu.sync_copy(zbuf, agg_sh.at[pl.ds(sid * 625 + t * 125, 125)])
    plsc.subcore_barrier()

    def chunk_body(ci, carry):
        off = base + ci * CHC
        row0 = off // SUB
        pltpu.sync_copy(src2_hbm.at[pl.ds(row0, 1)], sidx2)
        pltpu.sync_copy(dst2_hbm.at[pl.ds(row0, 1)], didx2)
        dsc = pltpu.async_copy(hc_hbm.at[sidx2.at[0]], hcrows, sem)
        pltpu.sync_copy(rad_hbm.at[pl.ds(off, CHC)], radc)
        dsc.wait()

        def edge(r, c2):
            for j in range(DD // 16):
                contrib[r, pl.ds(j * 16, 16)] = (
                    hcrows[r, pl.ds(j * 16, 16)]
                    * radc[r, pl.ds(j * 16, 16)])
            return c2
        lax.fori_loop(0, CHC, edge, 0)

        pltpu.sync_copy(contrib, agg_sh.at[didx2.at[0]], add=True)
        return carry

    lax.fori_loop(0, NCHC, chunk_body, 0)
    plsc.subcore_barrier()

    @pl.when(sid == 0)
    def _():
        pltpu.sync_copy(agg_sh, agg_hbm.at[cid])


def _tc_prologue_body(at_ref, emb_ref, win_ref, h_ref):
    at = at_ref[0, 0, :]
    onehot = (at[:, None]
              == lax.broadcasted_iota(jnp.int32, (1000, 100), 1)
              ).astype(jnp.float32)
    table = jnp.dot(emb_ref[...], win_ref[...],
                    preferred_element_type=jnp.float32)
    h_ref[...] = jnp.dot(onehot, table, preferred_element_type=jnp.float32)


def _tc_prologue(atomic3, emb, win):
    return pl.pallas_call(
        _tc_prologue_body,
        grid=(10,),
        in_specs=[
            pl.BlockSpec((1, 1, 1000), lambda i: (i, 0, 0)),
            pl.BlockSpec((100, 128), lambda i: (0, 0)),
            pl.BlockSpec((128, DD), lambda i: (0, 0)),
        ],
        out_specs=pl.BlockSpec((1000, DD), lambda i: (i, 0)),
        out_shape=jax.ShapeDtypeStruct((NN, DD), jnp.float32),
    )(atomic3, emb, win)


def _tc_rad_body(d2_ref, w1_ref, w2_ref, rad_ref):
    r = jnp.sqrt(d2_ref[0, 0, :] + 1e-8)
    a = jnp.maximum(r[:, None] * w1_ref[0, 0, :][None, :], 0.0)
    rad_ref[0] = jnp.dot(a, w2_ref[0], preferred_element_type=jnp.float32)


def _tc_rad(d2_3, wr1s, wr2s):
    return pl.pallas_call(
        _tc_rad_body,
        grid=(5, 80),
        in_specs=[
            pl.BlockSpec((1, 1, 4000), lambda i, j: (j, 0, 0)),
            pl.BlockSpec((1, 1, 16), lambda i, j: (i, 0, 0)),
            pl.BlockSpec((1, 16, DD), lambda i, j: (i, 0, 0)),
        ],
        out_specs=pl.BlockSpec((1, 4000, DD), lambda i, j: (i, j, 0)),
        out_shape=jax.ShapeDtypeStruct((5, EE, DD), jnp.float32),
    )(d2_3, wr1s, wr2s)


def _tc_proj_body(h_ref, wq_ref, wk_ref, wv_ref, q_ref, k_ref, v_ref):
    h = h_ref[...]
    q_ref[...] = jnp.dot(h, wq_ref[...], preferred_element_type=jnp.float32)
    k_ref[...] = jnp.dot(h, wk_ref[...], preferred_element_type=jnp.float32)
    v_ref[...] = jnp.dot(h, wv_ref[...], preferred_element_type=jnp.float32)


def _tc_proj(h, wq, wk, wv):
    blk = pl.BlockSpec((1000, DD), lambda i: (i, 0))
    wblk = pl.BlockSpec((DD, DD), lambda i: (0, 0))
    sds = jax.ShapeDtypeStruct((NN, DD), jnp.float32)
    return pl.pallas_call(
        _tc_proj_body,
        grid=(10,),
        in_specs=[blk, wblk, wblk, wblk],
        out_specs=(blk, blk, blk),
        out_shape=(sds, sds, sds),
    )(h, wq, wk, wv)


def _norm_update(h, aggb, grow, brow):
    numer = aggb[0, :, 0:DD] + aggb[1, :, 0:DD]
    den = aggb[0, :, DD:DD + 1] + aggb[1, :, DD:DD + 1]
    hn = h + numer / (den + 1e-9)
    nrm = jnp.abs(hn) + 1e-9
    return (hn / nrm) * jnp.maximum(nrm * grow + brow, 0.0)


def _tc_update_proj_body(h_ref, agg_ref, g_ref, b_ref,
                         wq_ref, wk_ref, wv_ref,
                         h_out, q_ref, k_ref, v_ref):
    hn = _norm_update(h_ref[...], agg_ref[...], g_ref[...], b_ref[...])
    h_out[...] = hn
    q_ref[...] = jnp.dot(hn, wq_ref[...], preferred_element_type=jnp.float32)
    k_ref[...] = jnp.dot(hn, wk_ref[...], preferred_element_type=jnp.float32)
    v_ref[...] = jnp.dot(hn, wv_ref[...], preferred_element_type=jnp.float32)


def _tc_update_proj(h, agg, grow, brow, wq, wk, wv):
    blk = pl.BlockSpec((1000, DD), lambda i: (i, 0))
    wblk = pl.BlockSpec((DD, DD), lambda i: (0, 0))
    gblk = pl.BlockSpec((1, DD), lambda i: (0, 0))
    ablk = pl.BlockSpec((NC, 1000, WROW), lambda i: (0, i, 0))
    sds = jax.ShapeDtypeStruct((NN, DD), jnp.float32)
    return pl.pallas_call(
        _tc_update_proj_body,
        grid=(10,),
        in_specs=[blk, ablk, gblk, gblk, wblk, wblk, wblk],
        out_specs=(blk, blk, blk, blk),
        out_shape=(sds, sds, sds, sds),
    )(h, agg, grow, brow, wq, wk, wv)


def _tc_update_proj2_body(h_ref, agg_ref, g_ref, b_ref, wc_ref, ws_ref,
                          hc_ref, hs_ref):
    hn = _norm_update(h_ref[...], agg_ref[...], g_ref[...], b_ref[...])
    hc_ref[...] = jnp.dot(hn, wc_ref[...], preferred_element_type=jnp.float32)
    hs_ref[...] = jnp.dot(hn, ws_ref[...], preferred_element_type=jnp.float32)


def _tc_update_proj2(h, agg, grow, brow, wc, wself):
    blk = pl.BlockSpec((1000, DD), lambda i: (i, 0))
    wblk = pl.BlockSpec((DD, DD), lambda i: (0, 0))
    gblk = pl.BlockSpec((1, DD), lambda i: (0, 0))
    ablk = pl.BlockSpec((NC, 1000, WROW), lambda i: (0, i, 0))
    sds = jax.ShapeDtypeStruct((NN, DD), jnp.float32)
    return pl.pallas_call(
        _tc_update_proj2_body,
        grid=(10,),
        in_specs=[blk, ablk, gblk, gblk, wblk, wblk],
        out_specs=(blk, blk),
        out_shape=(sds, sds),
    )(h, agg, grow, brow, wc, wself)


def _tc_final_body(conv_ref, hs_ref, w1_ref, b1_ref, w2_ref, b2_ref, out_ref):
    h2 = conv_ref[0] + conv_ref[1] + hs_ref[...]
    hp = jnp.mean(h2, axis=0, keepdims=True)
    z = jnp.maximum(
        jnp.dot(hp, w1_ref[...], preferred_element_type=jnp.float32)
        + b1_ref[...], 0.0)
    out_ref[...] = (jnp.dot(z, w2_ref[...],
                            preferred_element_type=jnp.float32)
                    + b2_ref[...])


def _tc_final(conv, hself, w1, b1, w2, b2):
    return pl.pallas_call(
        _tc_final_body,
        out_shape=jax.ShapeDtypeStruct((1, 1), jnp.float32),
    )(conv, hself, w1, b1, w2, b2)


def kernel(pos, atomic_numbers, edge_index, emb, Win, Wq, Wk, Wv, Wr1, Wr2,
           g, b, Wc, Wrc1, Wrc2, Wself, W1, b1, W2, b2):
    src = edge_index[0].astype(jnp.int32)
    dst = edge_index[1].astype(jnp.int32)
    src2 = src.reshape(EE // SUB, SUB)
    dst2 = dst.reshape(EE // SUB, SUB)
    atomic3 = atomic_numbers.astype(jnp.int32).reshape(10, 1, 1000)

    d2 = _sc_d2(pos[:, 0] + 0.0, pos[:, 1] + 0.0, pos[:, 2] + 0.0, src, dst)

    h = _tc_prologue(atomic3, emb, Win)

    wr1s = jnp.concatenate([Wr1[:, 0, :], Wrc1], axis=0).reshape(5, 1, 16)
    wr2s = jnp.concatenate([Wr2, Wrc2[None]], axis=0)
    rad_all = _tc_rad(d2.reshape(80, 1, 4000), wr1s, wr2s)

    agg = None
    for i in range(4):
        if i == 0:
            q, kf, vf = _tc_proj(h, Wq[0], Wk[0], Wv[0])
        else:
            h, q, kf, vf = _tc_update_proj(
                h, agg, g[i - 1].reshape(1, DD), b[i - 1].reshape(1, DD),
                Wq[i], Wk[i], Wv[i])
        logits, gmax = _sc_passA(q, kf, rad_all[i], src2, dst2)
        agg = _sc_passC(vf, rad_all[i], logits, gmax, src2, dst2)

    hc, hself = _tc_update_proj2(
        h, agg, g[3].reshape(1, DD), b[3].reshape(1, DD), Wc, Wself)
    conv = _sc_passD(hc, rad_all[4], src2, dst2)
    return _tc_final(conv, hself, W1, b1.reshape(1, DD), W2,
                     b2.reshape(1, 1))


# speedup vs baseline: 1.7417x; 1.0298x over previous
import functools

import jax
import jax.numpy as jnp
from jax import lax
from jax.experimental import pallas as pl
from jax.experimental.pallas import tpu as pltpu
from jax.experimental.pallas import tpu_sc as plsc

NN = 10000
EE = 320000
DD = 64
NC = 2
NS = 16
NW = NC * NS
EPT = EE // NW
CH = 400
NCHUNK = EPT // CH
SUB = 80
NSUB = CH // SUB
WROW = 80
CHC = 80
NCHC = EPT // CHC
NBUF = 5

_MESH = plsc.VectorSubcoreMesh(core_axis_name="c", subcore_axis_name="s")
_SC_PARAMS = pltpu.CompilerParams(use_tc_tiling_on_sc=False,
                                  needs_layout_passes=False)


def _wid():
    return lax.axis_index("s") * NC + lax.axis_index("c")


@functools.partial(
    pl.kernel,
    out_type=jax.ShapeDtypeStruct((EE,), jnp.float32),
    mesh=_MESH,
    compiler_params=_SC_PARAMS,
    scratch_types=[
        pltpu.VMEM((NN,), jnp.float32),
        pltpu.VMEM((NN,), jnp.float32),
        pltpu.VMEM((NN,), jnp.float32),
        pltpu.VMEM((CH,), jnp.int32),
        pltpu.VMEM((CH,), jnp.int32),
        pltpu.VMEM((CH,), jnp.float32),
    ],
)
def _sc_d2(px_hbm, py_hbm, pz_hbm, src_hbm, dst_hbm, d2_hbm,
           px, py, pz, sidx, didx, d2v):
    base = _wid() * EPT
    pltpu.sync_copy(px_hbm, px)
    pltpu.sync_copy(py_hbm, py)
    pltpu.sync_copy(pz_hbm, pz)

    def chunk_body(ci, carry):
        off = base + ci * CH
        pltpu.sync_copy(src_hbm.at[pl.ds(off, CH)], sidx)
        pltpu.sync_copy(dst_hbm.at[pl.ds(off, CH)], didx)

        def grp(gi, c2):
            s = sidx[pl.ds(gi * 16, 16)]
            d = didx[pl.ds(gi * 16, 16)]
            acc = jnp.zeros((16,), jnp.float32)
            for ref in (px, py, pz):
                dif = plsc.load_gather(ref, [d]) - plsc.load_gather(ref, [s])
                acc = acc + dif * dif
            d2v[pl.ds(gi * 16, 16)] = acc
            return c2

        lax.fori_loop(0, CH // 16, grp, 0)
        pltpu.sync_copy(d2v, d2_hbm.at[pl.ds(off, CH)])
        return carry

    lax.fori_loop(0, NCHUNK, chunk_body, 0)


@functools.partial(
    pl.kernel,
    out_type=(jax.ShapeDtypeStruct((EE,), jnp.float32),
              jax.ShapeDtypeStruct((NW, 16), jnp.float32)),
    mesh=_MESH,
    compiler_params=_SC_PARAMS,
    scratch_types=[
        pltpu.VMEM((NCHC, CHC), jnp.int32),
        pltpu.VMEM((NCHC, CHC), jnp.int32),
        pltpu.VMEM((NBUF, CHC, DD), jnp.float32),
        pltpu.VMEM((NBUF, CHC, DD), jnp.float32),
        pltpu.VMEM((NBUF, CHC, DD), jnp.float32),
        pltpu.VMEM((NBUF, CHC), jnp.float32),
        pltpu.VMEM((16,), jnp.float32),
        [pltpu.SemaphoreType.DMA] * NBUF,
    ],
)
def _sc_passA(q_hbm, k_hbm, rad_hbm, src2_hbm, dst2_hbm,
              logits_hbm, gmax_hbm,
              sidxall, didxall, qrows, krows, radc, lbuf, gmaxv, sems):
    wid = _wid()
    base = wid * EPT
    lane = lax.broadcasted_iota(jnp.int32, (16,), 0)
    pltpu.sync_copy(src2_hbm.at[pl.ds(wid * NCHC, NCHC)], sidxall)
    pltpu.sync_copy(dst2_hbm.at[pl.ds(wid * NCHC, NCHC)], didxall)

    def issue(ci, bb):
        off = base + ci * CHC
        pltpu.async_copy(q_hbm.at[didxall.at[ci]], qrows.at[bb], sems[bb])
        pltpu.async_copy(k_hbm.at[sidxall.at[ci]], krows.at[bb], sems[bb])
        pltpu.async_copy(rad_hbm.at[pl.ds(off, CHC)], radc.at[bb], sems[bb])

    for bb in range(NBUF):
        issue(bb, bb)

    def outer(gg, lmax):
        for bb in range(NBUF):
            ci = gg * NBUF + bb
            off = base + ci * CHC
            pltpu.make_async_copy(
                q_hbm.at[didxall.at[ci]], qrows.at[bb], sems[bb]).wait()
            pltpu.make_async_copy(
                k_hbm.at[sidxall.at[ci]], krows.at[bb], sems[bb]).wait()
            pltpu.make_async_copy(
                rad_hbm.at[pl.ds(off, CHC)], radc.at[bb], sems[bb]).wait()

            def grp(gi, lm):
                vs = []
                for t in range(16):
                    r = gi * 16 + t
                    acc = jnp.zeros((16,), jnp.float32)
                    for j in range(DD // 16):
                        acc = acc + (qrows[bb, r, pl.ds(j * 16, 16)]
                                     * krows[bb, r, pl.ds(j * 16, 16)]
                                     * radc[bb, r, pl.ds(j * 16, 16)])
                    vs.append(acc)
                dd = 1
                while len(vs) > 1:
                    nxt = []
                    mask = (lane & dd) != 0
                    perm = lane ^ dd

                    def _xperm(v, p=perm):
                        return lax.gather(
                            v, p[:, None],
                            lax.GatherDimensionNumbers(
                                offset_dims=(), collapsed_slice_dims=(0,),
                                start_index_map=(0,)),
                            (1,),
                            mode=lax.GatherScatterMode.PROMISE_IN_BOUNDS)
                    for ii in range(0, len(vs), 2):
                        xa = vs[ii] + _xperm(vs[ii])
                        xb = vs[ii + 1] + _xperm(vs[ii + 1])
                        nxt.append(jnp.where(mask, xb, xa))
                    vs = nxt
                    dd *= 2
                lgvec = vs[0] * 0.125
                lbuf[bb, pl.ds(gi * 16, 16)] = lgvec
                return jnp.maximum(lm, lgvec)

            lmax = lax.fori_loop(0, CHC // 16, grp, lmax)
            pltpu.sync_copy(lbuf.at[bb], logits_hbm.at[pl.ds(off, CHC)])

            @pl.when(ci + NBUF < NCHC)
            def _():
                issue(ci + NBUF, bb)
        return lmax

    lmax0 = jnp.full((16,), -3.0e38, jnp.float32)
    lmax = lax.fori_loop(0, NCHC // NBUF, outer, lmax0)
    gmaxv[...] = jnp.full((16,), jnp.max(lmax), jnp.float32)
    pltpu.sync_copy(gmaxv, gmax_hbm.at[wid])


@functools.partial(
    pl.kernel,
    out_type=jax.ShapeDtypeStruct((NC, NN, WROW), jnp.float32),
    mesh=_MESH,
    compiler_params=_SC_PARAMS,
    scratch_types=[
        pltpu.VMEM((1, CHC), jnp.int32),
        pltpu.VMEM((1, CHC), jnp.int32),
        pltpu.VMEM((CHC, DD), jnp.float32),
        pltpu.VMEM((CHC, DD), jnp.float32),
        pltpu.VMEM((CHC,), jnp.float32),
        pltpu.VMEM((CHC,), jnp.float32),
        pltpu.VMEM((CHC, WROW), jnp.float32),
        pltpu.VMEM((125, WROW), jnp.float32),
        pltpu.VMEM((NW, 16), jnp.float32),
        pltpu.VMEM_SHARED((NN, WROW), jnp.float32),
        pltpu.SemaphoreType.DMA,
    ],
)
def _sc_passC(v_hbm, rad_hbm, logits_hbm, gmax_hbm, src2_hbm, dst2_hbm,
              agg_hbm,
              sidx2, didx2, vrows, radc, lbuf, exbuf, contrib, zbuf, gmv,
              agg_sh, sem):
    cid = lax.axis_index("c")
    sid = lax.axis_index("s")
    wid = _wid()
    base = wid * EPT
    lane = lax.broadcasted_iota(jnp.int32, (16,), 0)
    zv = jnp.zeros((16,), jnp.float32)

    pltpu.sync_copy(gmax_hbm, gmv)
    mv = gmv[0]
    for kk in range(1, NW):
        mv = jnp.maximum(mv, gmv[kk])
    c = jnp.max(mv)

    def zrow(ri, carry):
        for j in range(WROW // 16):
            zbuf[ri, pl.ds(j * 16, 16)] = zv
        return carry
    lax.fori_loop(0, 125, zrow, 0)
    for t in range(5):
        pltpu.sync_copy(zbuf, agg_sh.at[pl.ds(sid * 625 + t * 125, 125)])
    plsc.subcore_barrier()

    def chunk_body(ci, carry):
        off = base + ci * CHC
        row0 = off // SUB
        pltpu.sync_copy(src2_hbm.at[pl.ds(row0, 1)], sidx2)
        pltpu.sync_copy(dst2_hbm.at[pl.ds(row0, 1)], didx2)
        dsc = pltpu.async_copy(v_hbm.at[sidx2.at[0]], vrows, sem)
        pltpu.sync_copy(rad_hbm.at[pl.ds(off, CHC)], radc)
        pltpu.sync_copy(logits_hbm.at[pl.ds(off, CHC)], lbuf)
        dsc.wait()

        def grp(gi, c2):
            exbuf[pl.ds(gi * 16, 16)] = jnp.exp(
                lbuf[pl.ds(gi * 16, 16)] - c)
            return c2
        lax.fori_loop(0, CHC // 16, grp, 0)

        def edge(r, c2):
            exv = plsc.load_gather(exbuf, [jnp.broadcast_to(r, (16,))])
            for j in range(DD // 16):
                contrib[r, pl.ds(j * 16, 16)] = (
                    exv * vrows[r, pl.ds(j * 16, 16)]
                    * radc[r, pl.ds(j * 16, 16)])
            contrib[r, pl.ds(DD, 16)] = jnp.where(lane == 0, exv, 0.0)
            return c2
        lax.fori_loop(0, CHC, edge, 0)

        pltpu.sync_copy(contrib, agg_sh.at[didx2.at[0]], add=True)
        return carry

    lax.fori_loop(0, NCHC, chunk_body, 0)
    plsc.subcore_barrier()

    @pl.when(sid == 0)
    def _():
        pltpu.sync_copy(agg_sh, agg_hbm.at[cid])


@functools.partial(
    pl.kernel,
    out_type=jax.ShapeDtypeStruct((NC, NN, DD), jnp.float32),
    mesh=_MESH,
    compiler_params=_SC_PARAMS,
    scratch_types=[
        pltpu.VMEM((1, CHC), jnp.int32),
        pltpu.VMEM((1, CHC), jnp.int32),
        pltpu.VMEM((CHC, DD), jnp.float32),
        pltpu.VMEM((CHC, DD), jnp.float32),
        pltpu.VMEM((CHC, DD), jnp.float32),
        pltpu.VMEM((125, DD), jnp.float32),
        pltpu.VMEM_SHARED((NN, DD), jnp.float32),
        pltpu.SemaphoreType.DMA,
    ],
)
def _sc_passD(hc_hbm, rad_hbm, src2_hbm, dst2_hbm, agg_hbm,
              sidx2, didx2, hcrows, radc, contrib, zbuf, agg_sh, sem):
    cid = lax.axis_index("c")
    sid = lax.axis_index("s")
    base = _wid() * EPT
    zv = jnp.zeros((16,), jnp.float32)

    def zrow(ri, carry):
        for j in range(DD // 16):
            zbuf[ri, pl.ds(j * 16, 16)] = zv
        return carry
    lax.fori_loop(0, 125, zrow, 0)
    for t in range(5):
        pltpu.sync_copy(zbuf, agg_sh.at[pl.ds(sid * 625 + t * 125, 125)])
    plsc.subcore_barrier()

    def chunk_body(ci, carry):
        off = base + ci * CHC
        row0 = off // SUB
        pltpu.sync_copy(src2_hbm.at[pl.ds(row0, 1)], sidx2)
        pltpu.sync_copy(dst2_hbm.at[pl.ds(row0, 1)], didx2)
        dsc = pltpu.async_copy(hc_hbm.at[sidx2.at[0]], hcrows, sem)
        pltpu.sync_copy(rad_hbm.at[pl.ds(off, CHC)], radc)
        dsc.wait()

        def edge(r, c2):
            for j in range(DD // 16):
                contrib[r, pl.ds(j * 16, 16)] = (
                    hcrows[r, pl.ds(j * 16, 16)]
                    * radc[r, pl.ds(j * 16, 16)])
            return c2
        lax.fori_loop(0, CHC, edge, 0)

        pltpu.sync_copy(contrib, agg_sh.at[didx2.at[0]], add=True)
        return carry

    lax.fori_loop(0, NCHC, chunk_body, 0)
    plsc.subcore_barrier()

    @pl.when(sid == 0)
    def _():
        pltpu.sync_copy(agg_sh, agg_hbm.at[cid])


def _tc_prologue_body(at_ref, emb_ref, win_ref, h_ref):
    at = at_ref[0, 0, :]
    onehot = (at[:, None]
              == lax.broadcasted_iota(jnp.int32, (1000, 100), 1)
              ).astype(jnp.float32)
    table = jnp.dot(emb_ref[...], win_ref[...],
                    preferred_element_type=jnp.float32)
    h_ref[...] = jnp.dot(onehot, table, preferred_element_type=jnp.float32)


def _tc_prologue(atomic3, emb, win):
    return pl.pallas_call(
        _tc_prologue_body,
        grid=(10,),
        in_specs=[
            pl.BlockSpec((1, 1, 1000), lambda i: (i, 0, 0)),
            pl.BlockSpec((100, 128), lambda i: (0, 0)),
            pl.BlockSpec((128, DD), lambda i: (0, 0)),
        ],
        out_specs=pl.BlockSpec((1000, DD), lambda i: (i, 0)),
        out_shape=jax.ShapeDtypeStruct((NN, DD), jnp.float32),
    )(atomic3, emb, win)


def _tc_rad_body(d2_ref, w1_ref, w2_ref, rad_ref):
    r = jnp.sqrt(d2_ref[0, 0, :] + 1e-8)
    a = jnp.maximum(r[:, None] * w1_ref[0, 0, :][None, :], 0.0)
    rad_ref[0] = jnp.dot(a, w2_ref[0], preferred_element_type=jnp.float32)


def _tc_rad(d2_3, wr1s, wr2s):
    return pl.pallas_call(
        _tc_rad_body,
        grid=(5, 80),
        in_specs=[
            pl.BlockSpec((1, 1, 4000), lambda i, j: (j, 0, 0)),
            pl.BlockSpec((1, 1, 16), lambda i, j: (i, 0, 0)),
            pl.BlockSpec((1, 16, DD), lambda i, j: (i, 0, 0)),
        ],
        out_specs=pl.BlockSpec((1, 4000, DD), lambda i, j: (i, j, 0)),
        out_shape=jax.ShapeDtypeStruct((5, EE, DD), jnp.float32),
    )(d2_3, wr1s, wr2s)


def _tc_proj_body(h_ref, wq_ref, wk_ref, wv_ref, q_ref, k_ref, v_ref):
    h = h_ref[...]
    q_ref[...] = jnp.dot(h, wq_ref[...], preferred_element_type=jnp.float32)
    k_ref[...] = jnp.dot(h, wk_ref[...], preferred_element_type=jnp.float32)
    v_ref[...] = jnp.dot(h, wv_ref[...], preferred_element_type=jnp.float32)


def _tc_proj(h, wq, wk, wv):
    blk = pl.BlockSpec((1000, DD), lambda i: (i, 0))
    wblk = pl.BlockSpec((DD, DD), lambda i: (0, 0))
    sds = jax.ShapeDtypeStruct((NN, DD), jnp.float32)
    return pl.pallas_call(
        _tc_proj_body,
        grid=(10,),
        in_specs=[blk, wblk, wblk, wblk],
        out_specs=(blk, blk, blk),
        out_shape=(sds, sds, sds),
    )(h, wq, wk, wv)


def _norm_update(h, aggb, grow, brow):
    numer = aggb[0, :, 0:DD] + aggb[1, :, 0:DD]
    den = aggb[0, :, DD:DD + 1] + aggb[1, :, DD:DD + 1]
    hn = h + numer / (den + 1e-9)
    nrm = jnp.abs(hn) + 1e-9
    return (hn / nrm) * jnp.maximum(nrm * grow + brow, 0.0)


def _tc_update_proj_body(h_ref, agg_ref, g_ref, b_ref,
                         wq_ref, wk_ref, wv_ref,
                         h_out, q_ref, k_ref, v_ref):
    hn = _norm_update(h_ref[...], agg_ref[...], g_ref[...], b_ref[...])
    h_out[...] = hn
    q_ref[...] = jnp.dot(hn, wq_ref[...], preferred_element_type=jnp.float32)
    k_ref[...] = jnp.dot(hn, wk_ref[...], preferred_element_type=jnp.float32)
    v_ref[...] = jnp.dot(hn, wv_ref[...], preferred_element_type=jnp.float32)


def _tc_update_proj(h, agg, grow, brow, wq, wk, wv):
    blk = pl.BlockSpec((1000, DD), lambda i: (i, 0))
    wblk = pl.BlockSpec((DD, DD), lambda i: (0, 0))
    gblk = pl.BlockSpec((1, DD), lambda i: (0, 0))
    ablk = pl.BlockSpec((NC, 1000, WROW), lambda i: (0, i, 0))
    sds = jax.ShapeDtypeStruct((NN, DD), jnp.float32)
    return pl.pallas_call(
        _tc_update_proj_body,
        grid=(10,),
        in_specs=[blk, ablk, gblk, gblk, wblk, wblk, wblk],
        out_specs=(blk, blk, blk, blk),
        out_shape=(sds, sds, sds, sds),
    )(h, agg, grow, brow, wq, wk, wv)


def _tc_update_proj2_body(h_ref, agg_ref, g_ref, b_ref, wc_ref, ws_ref,
                          hc_ref, hs_ref):
    hn = _norm_update(h_ref[...], agg_ref[...], g_ref[...], b_ref[...])
    hc_ref[...] = jnp.dot(hn, wc_ref[...], preferred_element_type=jnp.float32)
    hs_ref[...] = jnp.dot(hn, ws_ref[...], preferred_element_type=jnp.float32)


def _tc_update_proj2(h, agg, grow, brow, wc, wself):
    blk = pl.BlockSpec((1000, DD), lambda i: (i, 0))
    wblk = pl.BlockSpec((DD, DD), lambda i: (0, 0))
    gblk = pl.BlockSpec((1, DD), lambda i: (0, 0))
    ablk = pl.BlockSpec((NC, 1000, WROW), lambda i: (0, i, 0))
    sds = jax.ShapeDtypeStruct((NN, DD), jnp.float32)
    return pl.pallas_call(
        _tc_update_proj2_body,
        grid=(10,),
        in_specs=[blk, ablk, gblk, gblk, wblk, wblk],
        out_specs=(blk, blk),
        out_shape=(sds, sds),
    )(h, agg, grow, brow, wc, wself)


def _tc_final_body(conv_ref, hs_ref, w1_ref, b1_ref, w2_ref, b2_ref, out_ref):
    h2 = conv_ref[0] + conv_ref[1] + hs_ref[...]
    hp = jnp.mean(h2, axis=0, keepdims=True)
    z = jnp.maximum(
        jnp.dot(hp, w1_ref[...], preferred_element_type=jnp.float32)
        + b1_ref[...], 0.0)
    out_ref[...] = (jnp.dot(z, w2_ref[...],
                            preferred_element_type=jnp.float32)
                    + b2_ref[...])


def _tc_final(conv, hself, w1, b1, w2, b2):
    return pl.pallas_call(
        _tc_final_body,
        out_shape=jax.ShapeDtypeStruct((1, 1), jnp.float32),
    )(conv, hself, w1, b1, w2, b2)


def kernel(pos, atomic_numbers, edge_index, emb, Win, Wq, Wk, Wv, Wr1, Wr2,
           g, b, Wc, Wrc1, Wrc2, Wself, W1, b1, W2, b2):
    src = edge_index[0].astype(jnp.int32)
    dst = edge_index[1].astype(jnp.int32)
    src2 = src.reshape(EE // SUB, SUB)
    dst2 = dst.reshape(EE // SUB, SUB)
    atomic3 = atomic_numbers.astype(jnp.int32).reshape(10, 1, 1000)

    d2 = _sc_d2(pos[:, 0] + 0.0, pos[:, 1] + 0.0, pos[:, 2] + 0.0, src, dst)

    h = _tc_prologue(atomic3, emb, Win)

    wr1s = jnp.concatenate([Wr1[:, 0, :], Wrc1], axis=0).reshape(5, 1, 16)
    wr2s = jnp.concatenate([Wr2, Wrc2[None]], axis=0)
    rad_all = _tc_rad(d2.reshape(80, 1, 4000), wr1s, wr2s)

    agg = None
    for i in range(4):
        if i == 0:
            q, kf, vf = _tc_proj(h, Wq[0], Wk[0], Wv[0])
        else:
            h, q, kf, vf = _tc_update_proj(
                h, agg, g[i - 1].reshape(1, DD), b[i - 1].reshape(1, DD),
                Wq[i], Wk[i], Wv[i])
        logits, gmax = _sc_passA(q, kf, rad_all[i], src2, dst2)
        agg = _sc_passC(vf, rad_all[i], logits, gmax, src2, dst2)

    hc, hself = _tc_update_proj2(
        h, agg, g[3].reshape(1, DD), b[3].reshape(1, DD), Wc, Wself)
    conv = _sc_passD(hc, rad_all[4], src2, dst2)
    return _tc_final(conv, hself, W1, b1.reshape(1, DD), W2,
                     b2.reshape(1, 1))


# speedup vs baseline: 2.0111x; 1.1547x over previous
import functools

import jax
import jax.numpy as jnp
from jax import lax
from jax.experimental import pallas as pl
from jax.experimental.pallas import tpu as pltpu
from jax.experimental.pallas import tpu_sc as plsc

NN = 10000
EE = 320000
DD = 64
NC = 2
NS = 16
NW = NC * NS
EPT = EE // NW
CH = 400
NCHUNK = EPT // CH
SUB = 80
NSUB = CH // SUB
WROW = 80
CHC = 80
NCHC = EPT // CHC
NBUF = 5

_MESH = plsc.VectorSubcoreMesh(core_axis_name="c", subcore_axis_name="s")
_SC_PARAMS = pltpu.CompilerParams(use_tc_tiling_on_sc=False,
                                  needs_layout_passes=False)


def _wid():
    return lax.axis_index("s") * NC + lax.axis_index("c")


@functools.partial(
    pl.kernel,
    out_type=jax.ShapeDtypeStruct((EE,), jnp.float32),
    mesh=_MESH,
    compiler_params=_SC_PARAMS,
    scratch_types=[
        pltpu.VMEM((NN,), jnp.float32),
        pltpu.VMEM((NN,), jnp.float32),
        pltpu.VMEM((NN,), jnp.float32),
        pltpu.VMEM((CH,), jnp.int32),
        pltpu.VMEM((CH,), jnp.int32),
        pltpu.VMEM((CH,), jnp.float32),
    ],
)
def _sc_d2(px_hbm, py_hbm, pz_hbm, src_hbm, dst_hbm, d2_hbm,
           px, py, pz, sidx, didx, d2v):
    base = _wid() * EPT
    pltpu.sync_copy(px_hbm, px)
    pltpu.sync_copy(py_hbm, py)
    pltpu.sync_copy(pz_hbm, pz)

    def chunk_body(ci, carry):
        off = base + ci * CH
        pltpu.sync_copy(src_hbm.at[pl.ds(off, CH)], sidx)
        pltpu.sync_copy(dst_hbm.at[pl.ds(off, CH)], didx)

        def grp(gi, c2):
            s = sidx[pl.ds(gi * 16, 16)]
            d = didx[pl.ds(gi * 16, 16)]
            acc = jnp.zeros((16,), jnp.float32)
            for ref in (px, py, pz):
                dif = plsc.load_gather(ref, [d]) - plsc.load_gather(ref, [s])
                acc = acc + dif * dif
            d2v[pl.ds(gi * 16, 16)] = acc
            return c2

        lax.fori_loop(0, CH // 16, grp, 0)
        pltpu.sync_copy(d2v, d2_hbm.at[pl.ds(off, CH)])
        return carry

    lax.fori_loop(0, NCHUNK, chunk_body, 0)


@functools.partial(
    pl.kernel,
    out_type=(jax.ShapeDtypeStruct((EE,), jnp.float32),
              jax.ShapeDtypeStruct((NW, 16), jnp.float32)),
    mesh=_MESH,
    compiler_params=_SC_PARAMS,
    scratch_types=[
        pltpu.VMEM((NCHC, CHC), jnp.int32),
        pltpu.VMEM((NCHC, CHC), jnp.int32),
        pltpu.VMEM((NBUF, CHC, DD), jnp.float32),
        pltpu.VMEM((NBUF, CHC, DD), jnp.float32),
        pltpu.VMEM((NBUF, CHC, DD), jnp.float32),
        pltpu.VMEM((NBUF, CHC), jnp.float32),
        pltpu.VMEM((16,), jnp.float32),
        [pltpu.SemaphoreType.DMA] * NBUF,
    ],
)
def _sc_passA(q_hbm, k_hbm, rad_hbm, src2_hbm, dst2_hbm,
              logits_hbm, gmax_hbm,
              sidxall, didxall, qrows, krows, radc, lbuf, gmaxv, sems):
    wid = _wid()
    base = wid * EPT
    lane = lax.broadcasted_iota(jnp.int32, (16,), 0)
    pltpu.sync_copy(src2_hbm.at[pl.ds(wid * NCHC, NCHC)], sidxall)
    pltpu.sync_copy(dst2_hbm.at[pl.ds(wid * NCHC, NCHC)], didxall)

    def issue(ci, bb):
        off = base + ci * CHC
        pltpu.async_copy(q_hbm.at[didxall.at[ci]], qrows.at[bb], sems[bb])
        pltpu.async_copy(k_hbm.at[sidxall.at[ci]], krows.at[bb], sems[bb])
        pltpu.async_copy(rad_hbm.at[pl.ds(off, CHC)], radc.at[bb], sems[bb])

    for bb in range(NBUF):
        issue(bb, bb)

    def outer(gg, lmax):
        for bb in range(NBUF):
            ci = gg * NBUF + bb
            off = base + ci * CHC
            pltpu.make_async_copy(
                q_hbm.at[didxall.at[ci]], qrows.at[bb], sems[bb]).wait()
            pltpu.make_async_copy(
                k_hbm.at[sidxall.at[ci]], krows.at[bb], sems[bb]).wait()
            pltpu.make_async_copy(
                rad_hbm.at[pl.ds(off, CHC)], radc.at[bb], sems[bb]).wait()

            def grp(gi, lm):
                vs = []
                for t in range(16):
                    r = gi * 16 + t
                    acc = jnp.zeros((16,), jnp.float32)
                    for j in range(DD // 16):
                        acc = acc + (qrows[bb, r, pl.ds(j * 16, 16)]
                                     * krows[bb, r, pl.ds(j * 16, 16)]
                                     * radc[bb, r, pl.ds(j * 16, 16)])
                    vs.append(acc)
                dd = 1
                while len(vs) > 1:
                    nxt = []
                    mask = (lane & dd) != 0
                    perm = lane ^ dd

                    def _xperm(v, p=perm):
                        return lax.gather(
                            v, p[:, None],
                            lax.GatherDimensionNumbers(
                                offset_dims=(), collapsed_slice_dims=(0,),
                                start_index_map=(0,)),
                            (1,),
                            mode=lax.GatherScatterMode.PROMISE_IN_BOUNDS)
                    for ii in range(0, len(vs), 2):
                        xa = vs[ii] + _xperm(vs[ii])
                        xb = vs[ii + 1] + _xperm(vs[ii + 1])
                        nxt.append(jnp.where(mask, xb, xa))
                    vs = nxt
                    dd *= 2
                lgvec = vs[0] * 0.125
                lbuf[bb, pl.ds(gi * 16, 16)] = lgvec
                return jnp.maximum(lm, lgvec)

            lmax = lax.fori_loop(0, CHC // 16, grp, lmax)
            pltpu.sync_copy(lbuf.at[bb], logits_hbm.at[pl.ds(off, CHC)])

            @pl.when(ci + NBUF < NCHC)
            def _():
                issue(ci + NBUF, bb)
        return lmax

    lmax0 = jnp.full((16,), -3.0e38, jnp.float32)
    lmax = lax.fori_loop(0, NCHC // NBUF, outer, lmax0)
    gmaxv[...] = jnp.full((16,), jnp.max(lmax), jnp.float32)
    pltpu.sync_copy(gmaxv, gmax_hbm.at[wid])


@functools.partial(
    pl.kernel,
    out_type=jax.ShapeDtypeStruct((NC, NN, WROW), jnp.float32),
    mesh=_MESH,
    compiler_params=_SC_PARAMS,
    scratch_types=[
        pltpu.VMEM((NCHC, CHC), jnp.int32),
        pltpu.VMEM((NCHC, CHC), jnp.int32),
        pltpu.VMEM((CHC, DD), jnp.float32),
        pltpu.VMEM((CHC, DD), jnp.float32),
        pltpu.VMEM((CHC,), jnp.float32),
        pltpu.VMEM((CHC,), jnp.float32),
        pltpu.VMEM((CHC, WROW), jnp.float32),
        pltpu.VMEM((NW, 16), jnp.float32),
        pltpu.VMEM_SHARED((NN, WROW), jnp.float32),
        pltpu.SemaphoreType.DMA,
    ],
)
def _sc_passC(v_hbm, rad_hbm, logits_hbm, gmax_hbm, src2_hbm, dst2_hbm,
              agg_hbm,
              sidxall, didxall, vrows, radc, lbuf, exbuf, contrib, gmv,
              agg_sh, sem):
    cid = lax.axis_index("c")
    sid = lax.axis_index("s")
    wid = _wid()
    base = wid * EPT
    lane = lax.broadcasted_iota(jnp.int32, (16,), 0)
    zv = jnp.zeros((16,), jnp.float32)

    pltpu.sync_copy(gmax_hbm, gmv)
    mv = gmv[0]
    for kk in range(1, NW):
        mv = jnp.maximum(mv, gmv[kk])
    c = jnp.max(mv)

    pltpu.sync_copy(src2_hbm.at[pl.ds(wid * NCHC, NCHC)], sidxall)
    pltpu.sync_copy(dst2_hbm.at[pl.ds(wid * NCHC, NCHC)], didxall)

    def zrow(ri, carry):
        for j in range(WROW // 16):
            contrib[ri, pl.ds(j * 16, 16)] = zv
        return carry
    lax.fori_loop(0, CHC, zrow, 0)
    for t in range(7):
        pltpu.sync_copy(contrib,
                        agg_sh.at[pl.ds(sid * 625 + t * CHC, CHC)])
    pltpu.sync_copy(contrib.at[pl.ds(0, 65)],
                    agg_sh.at[pl.ds(sid * 625 + 560, 65)])
    plsc.subcore_barrier()

    def chunk_body(ci, carry):
        off = base + ci * CHC
        pltpu.async_copy(v_hbm.at[sidxall.at[ci]], vrows, sem)
        pltpu.async_copy(rad_hbm.at[pl.ds(off, CHC)], radc, sem)
        pltpu.async_copy(logits_hbm.at[pl.ds(off, CHC)], lbuf, sem)
        pltpu.make_async_copy(
            v_hbm.at[sidxall.at[ci]], vrows, sem).wait()
        pltpu.make_async_copy(
            rad_hbm.at[pl.ds(off, CHC)], radc, sem).wait()
        pltpu.make_async_copy(
            logits_hbm.at[pl.ds(off, CHC)], lbuf, sem).wait()

        def grp(gi, c2):
            exbuf[pl.ds(gi * 16, 16)] = jnp.exp(
                lbuf[pl.ds(gi * 16, 16)] - c)
            return c2
        lax.fori_loop(0, CHC // 16, grp, 0)

        def edge(r, c2):
            exv = plsc.load_gather(exbuf, [jnp.broadcast_to(r, (16,))])
            for j in range(DD // 16):
                contrib[r, pl.ds(j * 16, 16)] = (
                    exv * vrows[r, pl.ds(j * 16, 16)]
                    * radc[r, pl.ds(j * 16, 16)])
            contrib[r, pl.ds(DD, 16)] = jnp.where(lane == 0, exv, 0.0)
            return c2
        lax.fori_loop(0, CHC, edge, 0)

        pltpu.sync_copy(contrib, agg_sh.at[didxall.at[ci]], add=True)
        return carry

    lax.fori_loop(0, NCHC, chunk_body, 0)
    plsc.subcore_barrier()

    @pl.when(sid == 0)
    def _():
        pltpu.sync_copy(agg_sh, agg_hbm.at[cid])


@functools.partial(
    pl.kernel,
    out_type=jax.ShapeDtypeStruct((NC, NN, DD), jnp.float32),
    mesh=_MESH,
    compiler_params=_SC_PARAMS,
    scratch_types=[
        pltpu.VMEM((NCHC, CHC), jnp.int32),
        pltpu.VMEM((NCHC, CHC), jnp.int32),
        pltpu.VMEM((CHC, DD), jnp.float32),
        pltpu.VMEM((CHC, DD), jnp.float32),
        pltpu.VMEM((CHC, DD), jnp.float32),
        pltpu.VMEM_SHARED((NN, DD), jnp.float32),
        pltpu.SemaphoreType.DMA,
    ],
)
def _sc_passD(hc_hbm, rad_hbm, src2_hbm, dst2_hbm, agg_hbm,
              sidxall, didxall, hcrows, radc, contrib, agg_sh, sem):
    cid = lax.axis_index("c")
    sid = lax.axis_index("s")
    wid = _wid()
    base = wid * EPT
    zv = jnp.zeros((16,), jnp.float32)
    pltpu.sync_copy(src2_hbm.at[pl.ds(wid * NCHC, NCHC)], sidxall)
    pltpu.sync_copy(dst2_hbm.at[pl.ds(wid * NCHC, NCHC)], didxall)

    def zrow(ri, carry):
        for j in range(DD // 16):
            contrib[ri, pl.ds(j * 16, 16)] = zv
        return carry
    lax.fori_loop(0, CHC, zrow, 0)
    for t in range(7):
        pltpu.sync_copy(contrib,
                        agg_sh.at[pl.ds(sid * 625 + t * CHC, CHC)])
    pltpu.sync_copy(contrib.at[pl.ds(0, 65)],
                    agg_sh.at[pl.ds(sid * 625 + 560, 65)])
    plsc.subcore_barrier()

    def chunk_body(ci, carry):
        off = base + ci * CHC
        pltpu.async_copy(hc_hbm.at[sidxall.at[ci]], hcrows, sem)
        pltpu.async_copy(rad_hbm.at[pl.ds(off, CHC)], radc, sem)
        pltpu.make_async_copy(
            hc_hbm.at[sidxall.at[ci]], hcrows, sem).wait()
        pltpu.make_async_copy(
            rad_hbm.at[pl.ds(off, CHC)], radc, sem).wait()

        def edge(r, c2):
            for j in range(DD // 16):
                contrib[r, pl.ds(j * 16, 16)] = (
                    hcrows[r, pl.ds(j * 16, 16)]
                    * radc[r, pl.ds(j * 16, 16)])
            return c2
        lax.fori_loop(0, CHC, edge, 0)

        pltpu.sync_copy(contrib, agg_sh.at[didxall.at[ci]], add=True)
        return carry

    lax.fori_loop(0, NCHC, chunk_body, 0)
    plsc.subcore_barrier()

    @pl.when(sid == 0)
    def _():
        pltpu.sync_copy(agg_sh, agg_hbm.at[cid])


def _tc_prologue_body(at_ref, emb_ref, win_ref, h_ref):
    at = at_ref[0, 0, :]
    onehot = (at[:, None]
              == lax.broadcasted_iota(jnp.int32, (1000, 100), 1)
              ).astype(jnp.float32)
    table = jnp.dot(emb_ref[...], win_ref[...],
                    preferred_element_type=jnp.float32)
    h_ref[...] = jnp.dot(onehot, table, preferred_element_type=jnp.float32)


def _tc_prologue(atomic3, emb, win):
    return pl.pallas_call(
        _tc_prologue_body,
        grid=(10,),
        in_specs=[
            pl.BlockSpec((1, 1, 1000), lambda i: (i, 0, 0)),
            pl.BlockSpec((100, 128), lambda i: (0, 0)),
            pl.BlockSpec((128, DD), lambda i: (0, 0)),
        ],
        out_specs=pl.BlockSpec((1000, DD), lambda i: (i, 0)),
        out_shape=jax.ShapeDtypeStruct((NN, DD), jnp.float32),
    )(atomic3, emb, win)


def _tc_rad_body(d2_ref, w1_ref, w2_ref, rad_ref):
    r = jnp.sqrt(d2_ref[0, 0, :] + 1e-8)
    a = jnp.maximum(r[:, None] * w1_ref[0, 0, :][None, :], 0.0)
    rad_ref[0] = jnp.dot(a, w2_ref[0], preferred_element_type=jnp.float32)


def _tc_rad(d2_3, wr1s, wr2s):
    return pl.pallas_call(
        _tc_rad_body,
        grid=(5, 80),
        in_specs=[
            pl.BlockSpec((1, 1, 4000), lambda i, j: (j, 0, 0)),
            pl.BlockSpec((1, 1, 16), lambda i, j: (i, 0, 0)),
            pl.BlockSpec((1, 16, DD), lambda i, j: (i, 0, 0)),
        ],
        out_specs=pl.BlockSpec((1, 4000, DD), lambda i, j: (i, j, 0)),
        out_shape=jax.ShapeDtypeStruct((5, EE, DD), jnp.float32),
    )(d2_3, wr1s, wr2s)


def _tc_proj_body(h_ref, wq_ref, wk_ref, wv_ref, q_ref, k_ref, v_ref):
    h = h_ref[...]
    q_ref[...] = jnp.dot(h, wq_ref[...], preferred_element_type=jnp.float32)
    k_ref[...] = jnp.dot(h, wk_ref[...], preferred_element_type=jnp.float32)
    v_ref[...] = jnp.dot(h, wv_ref[...], preferred_element_type=jnp.float32)


def _tc_proj(h, wq, wk, wv):
    blk = pl.BlockSpec((1000, DD), lambda i: (i, 0))
    wblk = pl.BlockSpec((DD, DD), lambda i: (0, 0))
    sds = jax.ShapeDtypeStruct((NN, DD), jnp.float32)
    return pl.pallas_call(
        _tc_proj_body,
        grid=(10,),
        in_specs=[blk, wblk, wblk, wblk],
        out_specs=(blk, blk, blk),
        out_shape=(sds, sds, sds),
    )(h, wq, wk, wv)


def _norm_update(h, aggb, grow, brow):
    numer = aggb[0, :, 0:DD] + aggb[1, :, 0:DD]
    den = aggb[0, :, DD:DD + 1] + aggb[1, :, DD:DD + 1]
    hn = h + numer / (den + 1e-9)
    nrm = jnp.abs(hn) + 1e-9
    return (hn / nrm) * jnp.maximum(nrm * grow + brow, 0.0)


def _tc_update_proj_body(h_ref, agg_ref, g_ref, b_ref,
                         wq_ref, wk_ref, wv_ref,
                         h_out, q_ref, k_ref, v_ref):
    hn = _norm_update(h_ref[...], agg_ref[...], g_ref[...], b_ref[...])
    h_out[...] = hn
    q_ref[...] = jnp.dot(hn, wq_ref[...], preferred_element_type=jnp.float32)
    k_ref[...] = jnp.dot(hn, wk_ref[...], preferred_element_type=jnp.float32)
    v_ref[...] = jnp.dot(hn, wv_ref[...], preferred_element_type=jnp.float32)


def _tc_update_proj(h, agg, grow, brow, wq, wk, wv):
    blk = pl.BlockSpec((1000, DD), lambda i: (i, 0))
    wblk = pl.BlockSpec((DD, DD), lambda i: (0, 0))
    gblk = pl.BlockSpec((1, DD), lambda i: (0, 0))
    ablk = pl.BlockSpec((NC, 1000, WROW), lambda i: (0, i, 0))
    sds = jax.ShapeDtypeStruct((NN, DD), jnp.float32)
    return pl.pallas_call(
        _tc_update_proj_body,
        grid=(10,),
        in_specs=[blk, ablk, gblk, gblk, wblk, wblk, wblk],
        out_specs=(blk, blk, blk, blk),
        out_shape=(sds, sds, sds, sds),
    )(h, agg, grow, brow, wq, wk, wv)


def _tc_update_proj2_body(h_ref, agg_ref, g_ref, b_ref, wc_ref, ws_ref,
                          hc_ref, hs_ref):
    hn = _norm_update(h_ref[...], agg_ref[...], g_ref[...], b_ref[...])
    hc_ref[...] = jnp.dot(hn, wc_ref[...], preferred_element_type=jnp.float32)
    hs_ref[...] = jnp.dot(hn, ws_ref[...], preferred_element_type=jnp.float32)


def _tc_update_proj2(h, agg, grow, brow, wc, wself):
    blk = pl.BlockSpec((1000, DD), lambda i: (i, 0))
    wblk = pl.BlockSpec((DD, DD), lambda i: (0, 0))
    gblk = pl.BlockSpec((1, DD), lambda i: (0, 0))
    ablk = pl.BlockSpec((NC, 1000, WROW), lambda i: (0, i, 0))
    sds = jax.ShapeDtypeStruct((NN, DD), jnp.float32)
    return pl.pallas_call(
        _tc_update_proj2_body,
        grid=(10,),
        in_specs=[blk, ablk, gblk, gblk, wblk, wblk],
        out_specs=(blk, blk),
        out_shape=(sds, sds),
    )(h, agg, grow, brow, wc, wself)


def _tc_final_body(conv_ref, hs_ref, w1_ref, b1_ref, w2_ref, b2_ref, out_ref):
    h2 = conv_ref[0] + conv_ref[1] + hs_ref[...]
    hp = jnp.mean(h2, axis=0, keepdims=True)
    z = jnp.maximum(
        jnp.dot(hp, w1_ref[...], preferred_element_type=jnp.float32)
        + b1_ref[...], 0.0)
    out_ref[...] = (jnp.dot(z, w2_ref[...],
                            preferred_element_type=jnp.float32)
                    + b2_ref[...])


def _tc_final(conv, hself, w1, b1, w2, b2):
    return pl.pallas_call(
        _tc_final_body,
        out_shape=jax.ShapeDtypeStruct((1, 1), jnp.float32),
    )(conv, hself, w1, b1, w2, b2)


def kernel(pos, atomic_numbers, edge_index, emb, Win, Wq, Wk, Wv, Wr1, Wr2,
           g, b, Wc, Wrc1, Wrc2, Wself, W1, b1, W2, b2):
    src = edge_index[0].astype(jnp.int32)
    dst = edge_index[1].astype(jnp.int32)
    src2 = src.reshape(EE // SUB, SUB)
    dst2 = dst.reshape(EE // SUB, SUB)
    atomic3 = atomic_numbers.astype(jnp.int32).reshape(10, 1, 1000)

    d2 = _sc_d2(pos[:, 0] + 0.0, pos[:, 1] + 0.0, pos[:, 2] + 0.0, src, dst)

    h = _tc_prologue(atomic3, emb, Win)

    wr1s = jnp.concatenate([Wr1[:, 0, :], Wrc1], axis=0).reshape(5, 1, 16)
    wr2s = jnp.concatenate([Wr2, Wrc2[None]], axis=0)
    rad_all = _tc_rad(d2.reshape(80, 1, 4000), wr1s, wr2s)

    agg = None
    for i in range(4):
        if i == 0:
            q, kf, vf = _tc_proj(h, Wq[0], Wk[0], Wv[0])
        else:
            h, q, kf, vf = _tc_update_proj(
                h, agg, g[i - 1].reshape(1, DD), b[i - 1].reshape(1, DD),
                Wq[i], Wk[i], Wv[i])
        logits, gmax = _sc_passA(q, kf, rad_all[i], src2, dst2)
        agg = _sc_passC(vf, rad_all[i], logits, gmax, src2, dst2)

    hc, hself = _tc_update_proj2(
        h, agg, g[3].reshape(1, DD), b[3].reshape(1, DD), Wc, Wself)
    conv = _sc_passD(hc, rad_all[4], src2, dst2)
    return _tc_final(conv, hself, W1, b1.reshape(1, DD), W2,
                     b2.reshape(1, 1))
